# baseline (speedup 1.0000x reference)
import jax
import jax.numpy as jnp
from jax import lax
from jax.experimental import pallas as pl
from jax.experimental.pallas import tpu as pltpu
from jax.experimental.pallas import tpu_sc as plsc

_NC = 2
_NS = 16
_NW = _NC * _NS
_CHUNK = 128


def _sc_gather_body(uids, mids, gu_t, gm_t, mu_t, mm_t,
                    gu_o, gm_o, mu_o, mm_o,
                    uidx_v, midx_v, gu_v, gm_v, mu_v, mm_v, sem):
    bpw = uidx_v.shape[0]
    wid = lax.axis_index("s") * _NC + lax.axis_index("c")
    base = wid * bpw
    pltpu.sync_copy(uids.at[pl.ds(base, bpw)], uidx_v)
    pltpu.sync_copy(mids.at[pl.ds(base, bpw)], midx_v)
    copies = []
    for j in range(bpw // _CHUNK):
        sl = pl.ds(j * _CHUNK, _CHUNK)
        copies.append(pltpu.async_copy(gu_t.at[uidx_v.at[sl]], gu_v.at[sl], sem))
        copies.append(pltpu.async_copy(gm_t.at[midx_v.at[sl]], gm_v.at[sl], sem))
        copies.append(pltpu.async_copy(mu_t.at[uidx_v.at[sl]], mu_v.at[sl], sem))
        copies.append(pltpu.async_copy(mm_t.at[midx_v.at[sl]], mm_v.at[sl], sem))
    for cp in copies:
        cp.wait()
    pltpu.sync_copy(gu_v, gu_o.at[pl.ds(base, bpw)])
    pltpu.sync_copy(gm_v, gm_o.at[pl.ds(base, bpw)])
    pltpu.sync_copy(mu_v, mu_o.at[pl.ds(base, bpw)])
    pltpu.sync_copy(mm_v, mm_o.at[pl.ds(base, bpw)])


def _tc_mlp_body(gu_ref, gm_ref, mu_ref, mm_ref,
                 W1_ref, b1_ref, W2_ref, b2_ref, W3_ref, b3_ref,
                 Wo_ref, bo_ref, out_ref):
    x = jnp.concatenate([mu_ref[...], mm_ref[...]], axis=1)
    h = jnp.maximum(
        jnp.dot(x, W1_ref[...].T, preferred_element_type=jnp.float32)
        + b1_ref[...], 0.0)
    h = jnp.maximum(
        jnp.dot(h, W2_ref[...].T, preferred_element_type=jnp.float32)
        + b2_ref[...], 0.0)
    h = jnp.maximum(
        jnp.dot(h, W3_ref[...].T, preferred_element_type=jnp.float32)
        + b3_ref[...], 0.0)
    gmf = gu_ref[...] * gm_ref[...]
    comb = jnp.concatenate([gmf, h], axis=1)
    logit = jnp.sum(comb * Wo_ref[...], axis=1) + bo_ref[0, 0]
    out_ref[...] = jax.nn.sigmoid(logit)


def kernel(user_ids, movie_ids, gmf_user_emb, gmf_movie_emb,
           mlp_user_emb, mlp_movie_emb, W1, b1, W2, b2, W3, b3, Wo, bo):
    B = user_ids.shape[0]
    E = gmf_user_emb.shape[1]
    bpw = B // _NW

    mesh = plsc.VectorSubcoreMesh(core_axis_name="c", subcore_axis_name="s",
                                  num_cores=_NC, num_subcores=_NS)
    emb = jax.ShapeDtypeStruct((B, E), jnp.float32)
    sc_gather = pl.kernel(
        _sc_gather_body,
        out_type=(emb, emb, emb, emb),
        mesh=mesh,
        scratch_types=[
            pltpu.VMEM((bpw,), jnp.int32),
            pltpu.VMEM((bpw,), jnp.int32),
            pltpu.VMEM((bpw, E), jnp.float32),
            pltpu.VMEM((bpw, E), jnp.float32),
            pltpu.VMEM((bpw, E), jnp.float32),
            pltpu.VMEM((bpw, E), jnp.float32),
            pltpu.SemaphoreType.DMA,
        ],
        compiler_params=pltpu.CompilerParams(use_tc_tiling_on_sc=False),
    )
    gu_g, gm_g, mu_g, mm_g = sc_gather(
        user_ids, movie_ids, gmf_user_emb, gmf_movie_emb,
        mlp_user_emb, mlp_movie_emb)

    BLK = 2048
    d1, d_in = W1.shape
    d2 = W2.shape[0]
    d3 = W3.shape[0]
    row = pl.BlockSpec((BLK, E), lambda i: (i, 0))
    full = lambda shp: pl.BlockSpec(shp, lambda i: (0, 0))
    out = pl.pallas_call(
        _tc_mlp_body,
        grid=(B // BLK,),
        in_specs=[
            row, row, row, row,
            full((d1, d_in)), full((1, d1)),
            full((d2, d1)), full((1, d2)),
            full((d3, d2)), full((1, d3)),
            full((1, E + d3)), full((1, 1)),
        ],
        out_specs=pl.BlockSpec((BLK,), lambda i: (i,)),
        out_shape=jax.ShapeDtypeStruct((B,), jnp.float32),
    )(gu_g, gm_g, mu_g, mm_g,
      W1, b1.reshape(1, d1), W2, b2.reshape(1, d2), W3, b3.reshape(1, d3),
      Wo, bo.reshape(1, 1))
    return out
